# Initial kernel scaffold; baseline (speedup 1.0000x reference)
import jax
import jax.numpy as jnp
from jax.experimental import pallas as pl


def kernel(x, idx_a, idx_b, w0, w1, w2, w3):
    raise NotImplementedError("write your pallas kernel here")


# speedup vs baseline: 10.5562x; 10.5562x over previous
import functools

import numpy as np
import jax
import jax.numpy as jnp
from jax import lax
from jax.experimental import pallas as pl
from jax.experimental.pallas import tpu as pltpu
from jax.experimental.pallas import tpu_sc as plsc

B, C, H, W = 4, 32, 160, 160
K = 32
G0 = 8
OUT_H = OUT_W = 79
P = OUT_H * OUT_W
HQ = WQ = 80
PLANE = HQ * WQ
NPL = C * 4
P_PAD = PLANE

_OPS_M = np.array([
    [0, 0, 0, 0],
    [0, 0, 0, 1],
    [0, 1, 0, -1],
    [0, 1, 0, 0],
    [0, 0, 1, -1],
    [0, 0, 1, 0],
    [0, 1, 1, -2],
    [0, 1, 1, -1],
    [1, -1, -1, 1],
    [1, -1, -1, 2],
    [1, 0, -1, 0],
    [1, 0, -1, 1],
    [1, -1, 0, 0],
    [1, -1, 0, 1],
    [1, 0, 0, -1],
    [1, 0, 0, 0],
], np.float32)

_COL_BASES = (0, 16, 32, 48, 63)


def _sc_call(xp, meta_p, meta_s, coef):
    mesh = plsc.VectorSubcoreMesh(
        core_axis_name="c", subcore_axis_name="s", num_cores=2, num_subcores=16)

    @functools.partial(
        pl.kernel,
        mesh=mesh,
        compiler_params=pltpu.CompilerParams(
            use_tc_tiling_on_sc=False, needs_layout_passes=False),
        out_type=jax.ShapeDtypeStruct((B, K, P_PAD), jnp.float32),
        scratch_types=[
            pltpu.VMEM((16, PLANE), jnp.float32),
            pltpu.VMEM((P_PAD,), jnp.float32),
            pltpu.VMEM((16,), jnp.int32),
            pltpu.VMEM((16,), jnp.int32),
            pltpu.VMEM((64,), jnp.float32),
            pltpu.SemaphoreType.DMA,
        ],
    )
    def body(xp_hbm, mp_hbm, ms_hbm, cf_hbm, out_hbm, planes, outb, mpv, msv,
             cfv, sem):
        k = lax.axis_index("s") * 2 + lax.axis_index("c")
        pltpu.sync_copy(mp_hbm.at[k], mpv)
        pltpu.sync_copy(ms_hbm.at[k], msv)
        pltpu.sync_copy(cf_hbm.at[k], cfv)

        iota = lax.iota(jnp.int32, 16)
        pvec = mpv[...]
        svec = msv[...]

        def vbcast(v, n):
            idx = jnp.full((16, 1), n, jnp.int32)
            dnums = lax.GatherDimensionNumbers(
                offset_dims=(), collapsed_slice_dims=(0,), start_index_map=(0,))
            return lax.gather(v, idx, dnums, slice_sizes=(1,),
                              mode=lax.GatherScatterMode.PROMISE_IN_BOUNDS)

        shifts = [vbcast(svec, g) for g in range(16)]
        cvecs = [cfv[pl.ds(16 * j, 16)] for j in range(4)]
        cf = [[vbcast(cvecs[j], n) for j in range(4)] for n in range(15)]
        gsel = [jnp.full((16,), g, jnp.int32) for g in range(16)]

        def comb(a_, b_, cn):
            return cn[0] + cn[1] * a_ + cn[2] * b_ + cn[3] * (a_ * b_)

        for b in range(B):
            rows = pvec + np.int32(b * NPL)
            pltpu.async_copy(xp_hbm.at[rows], planes, sem).wait()

            def row_body(r, carry):
                rb = r * WQ
                ob = r * OUT_W
                for cb in _COL_BASES:
                    base = rb + cb + iota
                    vals = [plsc.load_gather(planes, [gsel[g], base + shifts[g]])
                            for g in range(16)]
                    t = [comb(vals[g], vals[8 + g], cf[g]) for g in range(G0)]
                    u = [comb(t[2 * j], t[2 * j + 1], cf[8 + j]) for j in range(4)]
                    v = [comb(u[2 * j], u[2 * j + 1], cf[12 + j]) for j in range(2)]
                    o = comb(v[0], v[1], cf[14])
                    plsc.store_scatter(outb, [ob + cb + iota], o)
                return carry

            lax.fori_loop(0, OUT_H, row_body, 0)
            pltpu.sync_copy(outb, out_hbm.at[b, k])

    return body(xp, meta_p, meta_s, coef)


def kernel(x, idx_a, idx_b, w0, w1, w2, w3):
    xp = (x.reshape(B, C, HQ, 2, WQ, 2)
           .transpose(0, 1, 3, 5, 2, 4)
           .reshape(B * NPL, PLANE))

    def side_meta(idx):
        dh, dw, ch = idx[0, :, :, 0], idx[0, :, :, 1], idx[0, :, :, 2]
        poff = ch * 4 + (dh % 2) * 2 + (dw % 2)
        shift = (dh // 2) * WQ + (dw // 2)
        return poff.astype(jnp.int32), shift.astype(jnp.int32)

    pa, sa = side_meta(idx_a)
    pb, sb = side_meta(idx_b)
    meta_p = jnp.concatenate([pa, pb], axis=1)
    meta_s = jnp.concatenate([sa, sb], axis=1)

    M = jnp.asarray(_OPS_M)
    coefs = [jnp.einsum('gki,ij->kjg', jax.nn.softmax(w, axis=-1), M,
                        precision=lax.Precision.HIGHEST)
             for w in (w0, w1, w2, w3)]
    coef = jnp.concatenate(coefs, axis=-1)
    coef = jnp.pad(coef, ((0, 0), (0, 0), (0, 1))).reshape(K, 64)

    out = _sc_call(xp, meta_p, meta_s, coef)
    return out[:, :, :P].reshape(B, K, OUT_H, OUT_W)


# speedup vs baseline: 12.8780x; 1.2200x over previous
import functools

import numpy as np
import jax
import jax.numpy as jnp
from jax import lax
from jax.experimental import pallas as pl
from jax.experimental.pallas import tpu as pltpu
from jax.experimental.pallas import tpu_sc as plsc

B, C, H, W = 4, 32, 160, 160
K = 32
G0 = 8
OUT_H = OUT_W = 79
P = OUT_H * OUT_W
HQ = WQ = 80
PLANE = HQ * WQ
NPL = C * 4
P_PAD = PLANE

_OPS_M = np.array([
    [0, 0, 0, 0],
    [0, 0, 0, 1],
    [0, 1, 0, -1],
    [0, 1, 0, 0],
    [0, 0, 1, -1],
    [0, 0, 1, 0],
    [0, 1, 1, -2],
    [0, 1, 1, -1],
    [1, -1, -1, 1],
    [1, -1, -1, 2],
    [1, 0, -1, 0],
    [1, 0, -1, 1],
    [1, -1, 0, 0],
    [1, -1, 0, 1],
    [1, 0, 0, -1],
    [1, 0, 0, 0],
], np.float32)

_COL_BASES = (0, 16, 32, 48, 63)


def _sc_call(xp, meta_p, meta_s, coef):
    mesh = plsc.VectorSubcoreMesh(
        core_axis_name="c", subcore_axis_name="s", num_cores=2, num_subcores=16)

    @functools.partial(
        pl.kernel,
        mesh=mesh,
        compiler_params=pltpu.CompilerParams(
            use_tc_tiling_on_sc=False, needs_layout_passes=False),
        out_type=jax.ShapeDtypeStruct((B, K, P_PAD), jnp.float32),
        scratch_types=[
            pltpu.VMEM((16, PLANE), jnp.float32),
            pltpu.VMEM((P_PAD,), jnp.float32),
            pltpu.VMEM((16,), jnp.int32),
            pltpu.VMEM((16,), jnp.int32),
            pltpu.VMEM((64,), jnp.float32),
            pltpu.SemaphoreType.DMA,
        ],
    )
    def body(xp_hbm, mp_hbm, ms_hbm, cf_hbm, out_hbm, planes, outb, mpv, msv,
             cfv, sem):
        k = lax.axis_index("s") * 2 + lax.axis_index("c")
        pltpu.sync_copy(mp_hbm.at[k], mpv)
        pltpu.sync_copy(ms_hbm.at[k], msv)
        pltpu.sync_copy(cf_hbm.at[k], cfv)

        pvec = mpv[...]
        svec = msv[...]
        shifts = [svec[g] for g in range(16)]
        cvecs = [cfv[pl.ds(16 * j, 16)] for j in range(4)]
        cf = [[cvecs[j][n] for j in range(4)] for n in range(15)]

        def comb(a_, b_, cn):
            return cn[0] + cn[1] * a_ + b_ * (cn[2] + cn[3] * a_)

        for b in range(B):
            rows = pvec + np.int32(b * NPL)
            pltpu.async_copy(xp_hbm.at[rows], planes, sem).wait()

            def row_body(r, carry):
                rb = r * WQ
                ob = r * OUT_W
                for cb in _COL_BASES:
                    vals = [planes[g, pl.ds(rb + cb + shifts[g], 16)]
                            for g in range(16)]
                    t = [comb(vals[g], vals[8 + g], cf[g]) for g in range(G0)]
                    u = [comb(t[2 * j], t[2 * j + 1], cf[8 + j]) for j in range(4)]
                    v = [comb(u[2 * j], u[2 * j + 1], cf[12 + j]) for j in range(2)]
                    outb[pl.ds(ob + cb, 16)] = comb(v[0], v[1], cf[14])
                return carry

            lax.fori_loop(0, OUT_H, row_body, 0)
            pltpu.sync_copy(outb, out_hbm.at[b, k])

    return body(xp, meta_p, meta_s, coef)


def kernel(x, idx_a, idx_b, w0, w1, w2, w3):
    xp = (x.reshape(B, C, HQ, 2, WQ, 2)
           .transpose(0, 1, 3, 5, 2, 4)
           .reshape(B * NPL, PLANE))

    def side_meta(idx):
        dh, dw, ch = idx[0, :, :, 0], idx[0, :, :, 1], idx[0, :, :, 2]
        poff = ch * 4 + (dh % 2) * 2 + (dw % 2)
        shift = (dh // 2) * WQ + (dw // 2)
        return poff.astype(jnp.int32), shift.astype(jnp.int32)

    pa, sa = side_meta(idx_a)
    pb, sb = side_meta(idx_b)
    meta_p = jnp.concatenate([pa, pb], axis=1)
    meta_s = jnp.concatenate([sa, sb], axis=1)

    M = jnp.asarray(_OPS_M)
    coefs = [jnp.einsum('gki,ij->kjg', jax.nn.softmax(w, axis=-1), M,
                        precision=lax.Precision.HIGHEST)
             for w in (w0, w1, w2, w3)]
    coef = jnp.concatenate(coefs, axis=-1)
    coef = jnp.pad(coef, ((0, 0), (0, 0), (0, 1))).reshape(K, 64)

    out = _sc_call(xp, meta_p, meta_s, coef)
    return out[:, :, :P].reshape(B, K, OUT_H, OUT_W)


# speedup vs baseline: 13.2405x; 1.0281x over previous
import functools

import numpy as np
import jax
import jax.numpy as jnp
from jax import lax
from jax.experimental import pallas as pl
from jax.experimental.pallas import tpu as pltpu
from jax.experimental.pallas import tpu_sc as plsc

B, C, H, W = 4, 32, 160, 160
K = 32
G0 = 8
OUT_H = OUT_W = 79
P = OUT_H * OUT_W
HQ = WQ = 80
PLANE = HQ * WQ
NPL = C * 4

_OPS_M = np.array([
    [0, 0, 0, 0],
    [0, 0, 0, 1],
    [0, 1, 0, -1],
    [0, 1, 0, 0],
    [0, 0, 1, -1],
    [0, 0, 1, 0],
    [0, 1, 1, -2],
    [0, 1, 1, -1],
    [1, -1, -1, 1],
    [1, -1, -1, 2],
    [1, 0, -1, 0],
    [1, 0, -1, 1],
    [1, -1, 0, 0],
    [1, -1, 0, 1],
    [1, 0, 0, -1],
    [1, 0, 0, 0],
], np.float32)

_COL_BASES = (0, 16, 32, 48, 63)


def _sc_call(xp, meta_p, meta_s, coef):
    mesh = plsc.VectorSubcoreMesh(
        core_axis_name="c", subcore_axis_name="s", num_cores=2, num_subcores=16)

    @functools.partial(
        pl.kernel,
        mesh=mesh,
        compiler_params=pltpu.CompilerParams(
            use_tc_tiling_on_sc=False, needs_layout_passes=False),
        out_type=jax.ShapeDtypeStruct((B, K, P), jnp.float32),
        scratch_types=[
            pltpu.VMEM((16, PLANE), jnp.float32),
            pltpu.VMEM((P,), jnp.float32),
            pltpu.VMEM((16,), jnp.int32),
            pltpu.VMEM((16,), jnp.int32),
            pltpu.VMEM((64,), jnp.float32),
            pltpu.SemaphoreType.DMA,
        ],
    )
    def body(xp_hbm, mp_hbm, ms_hbm, cf_hbm, out_hbm, planes, outb, mpv, msv,
             cfv, sem):
        k = lax.axis_index("s") * 2 + lax.axis_index("c")
        pltpu.sync_copy(mp_hbm.at[k], mpv)
        pltpu.sync_copy(ms_hbm.at[k], msv)
        pltpu.sync_copy(cf_hbm.at[k], cfv)

        pvec = mpv[...]
        svec = msv[...]
        shifts = [svec[g] for g in range(16)]
        cvecs = [cfv[pl.ds(16 * j, 16)] for j in range(4)]
        cf = [[cvecs[j][n] for j in range(4)] for n in range(15)]

        def comb(a_, b_, cn):
            return cn[0] + cn[1] * a_ + b_ * (cn[2] + cn[3] * a_)

        for b in range(B):
            rows = pvec + np.int32(b * NPL)
            pltpu.async_copy(xp_hbm.at[rows], planes, sem).wait()

            @plsc.parallel_loop(0, OUT_H, unroll=2)
            def row_body(r):
                rb = r * WQ
                ob = r * OUT_W
                for cb in _COL_BASES:
                    vals = [planes[g, pl.ds(rb + cb + shifts[g], 16)]
                            for g in range(16)]
                    t = [comb(vals[g], vals[8 + g], cf[g]) for g in range(G0)]
                    u = [comb(t[2 * j], t[2 * j + 1], cf[8 + j]) for j in range(4)]
                    v = [comb(u[2 * j], u[2 * j + 1], cf[12 + j]) for j in range(2)]
                    outb[pl.ds(ob + cb, 16)] = comb(v[0], v[1], cf[14])

            pltpu.sync_copy(outb, out_hbm.at[b, k])

    return body(xp, meta_p, meta_s, coef)


def kernel(x, idx_a, idx_b, w0, w1, w2, w3):
    xp = (x.reshape(B, C, HQ, 2, WQ, 2)
           .transpose(0, 1, 3, 5, 2, 4)
           .reshape(B * NPL, PLANE))

    def side_meta(idx):
        dh, dw, ch = idx[0, :, :, 0], idx[0, :, :, 1], idx[0, :, :, 2]
        poff = ch * 4 + (dh % 2) * 2 + (dw % 2)
        shift = (dh // 2) * WQ + (dw // 2)
        return poff.astype(jnp.int32), shift.astype(jnp.int32)

    pa, sa = side_meta(idx_a)
    pb, sb = side_meta(idx_b)
    meta_p = jnp.concatenate([pa, pb], axis=1)
    meta_s = jnp.concatenate([sa, sb], axis=1)

    M = jnp.asarray(_OPS_M)
    coefs = [jnp.einsum('gki,ij->kjg', jax.nn.softmax(w, axis=-1), M,
                        precision=lax.Precision.HIGHEST)
             for w in (w0, w1, w2, w3)]
    coef = jnp.concatenate(coefs, axis=-1)
    coef = jnp.pad(coef, ((0, 0), (0, 0), (0, 1))).reshape(K, 64)

    out = _sc_call(xp, meta_p, meta_s, coef)
    return out.reshape(B, K, OUT_H, OUT_W)
